# baseline (speedup 1.0000x reference)
import functools

import jax
import jax.numpy as jnp
from jax import lax
from jax.experimental import pallas as pl
from jax.experimental.pallas import tpu as pltpu
from jax.experimental.pallas import tpu_sc as plsc

N = 10000
E = 160000
D = 256
DH = 128
NCORE = 2
NSUB = 16
LANES = 16
EP = 163840
EPT = EP // NSUB
SC = 1024
HC = 512
NPAD = 10240
RPT = NPAD // NSUB

_f32 = jnp.float32
_i32 = jnp.int32


def _mesh():
    return plsc.VectorSubcoreMesh(core_axis_name="c", subcore_axis_name="s",
                                  num_cores=NCORE, num_subcores=NSUB)


def _deg_body(src2, dst2, w16, zr16, ds_out, dd_out,
              ds_acc, dd_acc, zb, wrow, sidx, didx):
    c = lax.axis_index("c")
    s = lax.axis_index("s")

    pltpu.sync_copy(zr16, zb)

    def _zfill(g, _):
        r0 = pl.multiple_of(s * RPT + g * 64, 64)
        pltpu.sync_copy(zb, ds_acc.at[pl.ds(r0, 64)])
        pltpu.sync_copy(zb, dd_acc.at[pl.ds(r0, 64)])
        return _
    lax.fori_loop(0, RPT // 64, _zfill, 0)
    plsc.subcore_barrier()

    hpt = EPT // 2

    def _chunk(g, _):
        row0 = pl.multiple_of(c * (EP // 256) + s * (hpt // 128) + g * 8, 8)
        pltpu.sync_copy(src2.at[pl.ds(row0, 8)], sidx)
        pltpu.sync_copy(dst2.at[pl.ds(row0, 8)], didx)
        for k in range(8):
            er0 = pl.multiple_of(c * (EP // 2) + s * hpt + g * SC + k * 128,
                                 128)
            pltpu.sync_copy(w16.at[pl.ds(er0, 128)], wrow)
            pltpu.sync_copy(wrow, ds_acc.at[sidx.at[k]], add=True)
            pltpu.sync_copy(wrow, dd_acc.at[didx.at[k]], add=True)
        return _
    lax.fori_loop(0, hpt // SC, _chunk, 0)
    plsc.subcore_barrier()

    r0 = pl.multiple_of(s * RPT, 64)
    pltpu.sync_copy(ds_acc.at[pl.ds(r0, RPT)], ds_out.at[c].at[pl.ds(r0, RPT)])
    pltpu.sync_copy(dd_acc.at[pl.ds(r0, RPT)], dd_out.at[c].at[pl.ds(r0, RPT)])


def _make_deg_kernel():
    return pl.kernel(
        _deg_body,
        out_type=(jax.ShapeDtypeStruct((NCORE, NPAD, 16), _f32),
                  jax.ShapeDtypeStruct((NCORE, NPAD, 16), _f32)),
        mesh=_mesh(),
        compiler_params=pltpu.CompilerParams(use_tc_tiling_on_sc=False),
        scratch_types=[
            pltpu.VMEM_SHARED((NPAD, 16), _f32),
            pltpu.VMEM_SHARED((NPAD, 16), _f32),
            pltpu.VMEM((64, 16), _f32),
            pltpu.VMEM((128, 16), _f32),
            pltpu.VMEM((8, 128), _i32),
            pltpu.VMEM((8, 128), _i32),
        ],
    )


def _msg_body(h2, src2, dst2, wb, zrows, out,
              acc, rows_a, rows_b, sidx, didx, wc_a, wc_b, zb,
              sem_ga, sem_gb, sem_sa, sem_sb, sem_wa, sem_wb):
    c = lax.axis_index("c")
    s = lax.axis_index("s")

    pltpu.sync_copy(zrows, zb)

    def _zfill(g, _):
        r0 = pl.multiple_of(s * RPT + g * 32, 32)
        pltpu.sync_copy(zb, acc.at[pl.ds(r0, 32)])
        return _
    lax.fori_loop(0, RPT // 32, _zfill, 0)
    plsc.subcore_barrier()

    def _scale(buf, wc, base):
        def body(r8, _2):
            wr = jnp.int32(base) + r8
            for i in range(8):
                nb = wc[wr, pl.ds(i * LANES, LANES)]
                er = r8 * 8 + i
                for q in range(8):
                    seg = buf[er, pl.ds(q * LANES, LANES)]
                    buf[er, pl.ds(q * LANES, LANES)] = seg * nb
            return _2
        lax.fori_loop(0, 16, body, 0)

    gbufs = (rows_a, rows_b)
    gsems = (sem_ga, sem_gb)
    ssems = (sem_sa, sem_sb)
    wcs = (wc_a, wc_b)
    wsems = (sem_wa, sem_wb)

    def _chunk(g, _):
        row0 = pl.multiple_of(s * (EPT // 128) + g * 8, 8)
        pltpu.sync_copy(src2.at[pl.ds(row0, 8)], sidx)
        pltpu.sync_copy(dst2.at[pl.ds(row0, 8)], didx)
        wrow0 = pl.multiple_of(s * (EPT // 8) + g * 128, 8)
        wpend = [pltpu.async_copy(wb.at[pl.ds(wrow0, 32)], wc_a, sem_wa),
                 None]
        gpend = [pltpu.async_copy(h2.at[c].at[sidx.at[0]], rows_a, sem_ga),
                 None]
        spend = [None, None]
        for k in range(8):
            b = k % 2
            p = k // 2
            if k % 2 == 0:
                wpend[p % 2].wait()
                if p < 3:
                    wr1 = pl.multiple_of(wrow0 + (p + 1) * 32, 8)
                    wpend[(p + 1) % 2] = pltpu.async_copy(
                        wb.at[pl.ds(wr1, 32)], wcs[(p + 1) % 2],
                        wsems[(p + 1) % 2])
            gpend[b].wait()
            if k < 7:
                nb_ = (k + 1) % 2
                if spend[nb_] is not None:
                    spend[nb_].wait()
                gpend[nb_] = pltpu.async_copy(
                    h2.at[c].at[sidx.at[k + 1]], gbufs[nb_], gsems[nb_])
            _scale(gbufs[b], wcs[p % 2], (k % 2) * 16)
            spend[b] = pltpu.async_copy(gbufs[b], acc.at[didx.at[k]],
                                        ssems[b], add=True)
        spend[0].wait()
        spend[1].wait()
        return _
    lax.fori_loop(0, EPT // SC, _chunk, 0)
    plsc.subcore_barrier()

    nfull = N // RPT
    rem = N - nfull * RPT

    @pl.when(s < nfull)
    def _():
        r0 = pl.multiple_of(s * RPT, 64)
        pltpu.sync_copy(acc.at[pl.ds(r0, RPT)], out.at[c].at[pl.ds(r0, RPT)])

    @pl.when(s == nfull)
    def _():
        pltpu.sync_copy(acc.at[pl.ds(nfull * RPT, rem)],
                        out.at[c].at[pl.ds(nfull * RPT, rem)])


def _make_msg_kernel():
    return pl.kernel(
        _msg_body,
        out_type=jax.ShapeDtypeStruct((NCORE, N, DH), _f32),
        mesh=_mesh(),
        scratch_types=[
            pltpu.VMEM_SHARED((NPAD, DH), _f32),
            pltpu.VMEM((128, DH), _f32),
            pltpu.VMEM((128, DH), _f32),
            pltpu.VMEM((8, 128), _i32),
            pltpu.VMEM((8, 128), _i32),
            pltpu.VMEM((32, 128), _f32),
            pltpu.VMEM((32, 128), _f32),
            pltpu.VMEM((32, DH), _f32),
            pltpu.SemaphoreType.DMA,
            pltpu.SemaphoreType.DMA,
            pltpu.SemaphoreType.DMA,
            pltpu.SemaphoreType.DMA,
            pltpu.SemaphoreType.DMA,
            pltpu.SemaphoreType.DMA,
        ],
    )


def _rs_of(dref):
    return lax.rsqrt(jnp.maximum(dref[...], 1e-6))


def _t0_body(x_ref, w_ref, dsr_ref, ddr_ref, o_ref, os_ref, od_ref):
    dsv = dsr_ref[0, :N, 0:1] + dsr_ref[1, :N, 0:1]
    ddv = ddr_ref[0, :N, 0:1] + ddr_ref[1, :N, 0:1]
    os_ref[...] = dsv
    od_ref[...] = ddv
    h = jnp.dot(x_ref[...], w_ref[...], preferred_element_type=_f32)
    h = h * lax.rsqrt(jnp.maximum(dsv, 1e-6))
    o_ref[0] = h[:, :DH]
    o_ref[1] = h[:, DH:]


def _tmid_body(emit_xin, a_ref, xin_ref, g_ref, be_ref, b_ref, w_ref,
               dsv_ref, ddv_ref, o_ref, *maybe_xin_out):
    rd = _rs_of(ddv_ref)
    rs = _rs_of(dsv_ref)
    acc = None
    cs = []
    for half in range(2):
        ch = a_ref[half] * rd + b_ref[half]
        cs.append(ch)
    for half in range(2):
        xin = cs[half] if emit_xin else xin_ref[half]
        t = jax.nn.relu(cs[half])
        mu = jnp.mean(t, axis=0, keepdims=True)
        var = jnp.mean((t - mu) ** 2, axis=0, keepdims=True)
        bn = g_ref[half] * (t - mu) / jnp.sqrt(var + 1e-5) + be_ref[half]
        u = xin + bn
        w = w_ref[pl.ds(half * DH, DH), :]
        part = jnp.dot(u, w, preferred_element_type=_f32)
        acc = part if acc is None else acc + part
    acc = acc * rs
    o_ref[0] = acc[:, :DH]
    o_ref[1] = acc[:, DH:]
    if emit_xin:
        xo_ref = maybe_xin_out[0]
        xo_ref[0] = cs[0]
        xo_ref[1] = cs[1]


def _tfin_body(a_ref, ddv_ref, b_ref, o_ref):
    rd = _rs_of(ddv_ref)
    o_ref[...] = jnp.concatenate(
        [a_ref[0] * rd + b_ref[0], a_ref[1] * rd + b_ref[1]], axis=1)


_TC_PARAMS = pltpu.CompilerParams(vmem_limit_bytes=100 * 1024 * 1024)

_t0 = pl.pallas_call(
    _t0_body, out_shape=(jax.ShapeDtypeStruct((NCORE, N, DH), _f32),
                         jax.ShapeDtypeStruct((N, 1), _f32),
                         jax.ShapeDtypeStruct((N, 1), _f32)),
    compiler_params=_TC_PARAMS)

_tmid0 = pl.pallas_call(
    functools.partial(_tmid_body, True),
    out_shape=(jax.ShapeDtypeStruct((NCORE, N, DH), _f32),
               jax.ShapeDtypeStruct((NCORE, N, DH), _f32)),
    compiler_params=_TC_PARAMS)

_tmid1 = pl.pallas_call(
    functools.partial(_tmid_body, False),
    out_shape=jax.ShapeDtypeStruct((NCORE, N, DH), _f32),
    compiler_params=_TC_PARAMS)

_tfin = pl.pallas_call(
    _tfin_body, out_shape=jax.ShapeDtypeStruct((N, D), _f32),
    compiler_params=_TC_PARAMS)


def kernel(x, edge_index, edge_attr, W0, b0, W1, b1, W2, b2,
           gamma0, beta0, gamma1, beta1):
    src = edge_index[0].astype(_i32)
    dst = edge_index[1].astype(_i32)
    pad = EP - E
    w_pad = jnp.pad(edge_attr, (0, pad))
    src2 = jnp.pad(src, (0, pad)).reshape(EP // 128, 128)
    dst2 = jnp.pad(dst, (0, pad)).reshape(EP // 128, 128)
    w16 = jnp.broadcast_to(w_pad[:, None], (EP, 16))
    wb = jnp.broadcast_to(w_pad[:, None], (EP, 16)).reshape(EP // 8, 128)
    zr16 = jnp.zeros((64, 16), _f32)
    zrows = jnp.zeros((32, DH), _f32)

    deg_k = _make_deg_kernel()
    msg_k = _make_msg_kernel()

    ds_raw, dd_raw = deg_k(src2, dst2, w16, zr16)
    b0_2 = b0.reshape(2, 1, DH)
    b1_2 = b1.reshape(2, 1, DH)
    b2_2 = b2.reshape(2, 1, DH)
    g0_2 = gamma0.reshape(2, 1, DH)
    be0_2 = beta0.reshape(2, 1, DH)
    g1_2 = gamma1.reshape(2, 1, DH)
    be1_2 = beta1.reshape(2, 1, DH)

    h0, dsv, ddv = _t0(x, W0, ds_raw, dd_raw)
    a0 = msg_k(h0, src2, dst2, wb, zrows)
    h1, c0 = _tmid0(a0, a0, g0_2, be0_2, b0_2, W1, dsv, ddv)
    a1 = msg_k(h1, src2, dst2, wb, zrows)
    h2 = _tmid1(a1, c0, g1_2, be1_2, b1_2, W2, dsv, ddv)
    a2 = msg_k(h2, src2, dst2, wb, zrows)
    out = _tfin(a2, ddv, b2_2)
    return out


# speedup vs baseline: 1.0165x; 1.0165x over previous
import functools

import jax
import jax.numpy as jnp
from jax import lax
from jax.experimental import pallas as pl
from jax.experimental.pallas import tpu as pltpu
from jax.experimental.pallas import tpu_sc as plsc

N = 10000
E = 160000
D = 256
DH = 128
NCORE = 2
NSUB = 16
LANES = 16
EP = 163840
EPT = EP // NSUB
SC = 1024
HC = 512
NPAD = 10240
RPT = NPAD // NSUB

_f32 = jnp.float32
_i32 = jnp.int32


def _mesh():
    return plsc.VectorSubcoreMesh(core_axis_name="c", subcore_axis_name="s",
                                  num_cores=NCORE, num_subcores=NSUB)


def _deg_body(src2, dst2, w16, zr16, ds_out, dd_out,
              ds_acc, dd_acc, zb, wrow, sidx, didx):
    c = lax.axis_index("c")
    s = lax.axis_index("s")

    pltpu.sync_copy(zr16, zb)

    def _zfill(g, _):
        r0 = pl.multiple_of(s * RPT + g * 64, 64)
        pltpu.sync_copy(zb, ds_acc.at[pl.ds(r0, 64)])
        pltpu.sync_copy(zb, dd_acc.at[pl.ds(r0, 64)])
        return _
    lax.fori_loop(0, RPT // 64, _zfill, 0)
    plsc.subcore_barrier()

    hpt = EPT // 2

    def _chunk(g, _):
        row0 = pl.multiple_of(c * (EP // 256) + s * (hpt // 128) + g * 8, 8)
        pltpu.sync_copy(src2.at[pl.ds(row0, 8)], sidx)
        pltpu.sync_copy(dst2.at[pl.ds(row0, 8)], didx)
        for k in range(8):
            er0 = pl.multiple_of(c * (EP // 2) + s * hpt + g * SC + k * 128,
                                 128)
            pltpu.sync_copy(w16.at[pl.ds(er0, 128)], wrow)
            pltpu.sync_copy(wrow, ds_acc.at[sidx.at[k]], add=True)
            pltpu.sync_copy(wrow, dd_acc.at[didx.at[k]], add=True)
        return _
    lax.fori_loop(0, hpt // SC, _chunk, 0)
    plsc.subcore_barrier()

    r0 = pl.multiple_of(s * RPT, 64)
    pltpu.sync_copy(ds_acc.at[pl.ds(r0, RPT)], ds_out.at[c].at[pl.ds(r0, RPT)])
    pltpu.sync_copy(dd_acc.at[pl.ds(r0, RPT)], dd_out.at[c].at[pl.ds(r0, RPT)])


def _make_deg_kernel():
    return pl.kernel(
        _deg_body,
        out_type=(jax.ShapeDtypeStruct((NCORE, NPAD, 16), _f32),
                  jax.ShapeDtypeStruct((NCORE, NPAD, 16), _f32)),
        mesh=_mesh(),
        compiler_params=pltpu.CompilerParams(use_tc_tiling_on_sc=False),
        scratch_types=[
            pltpu.VMEM_SHARED((NPAD, 16), _f32),
            pltpu.VMEM_SHARED((NPAD, 16), _f32),
            pltpu.VMEM((64, 16), _f32),
            pltpu.VMEM((128, 16), _f32),
            pltpu.VMEM((8, 128), _i32),
            pltpu.VMEM((8, 128), _i32),
        ],
    )


def _msg_body(h2, src2, dst2, wb, zrows, out,
              acc, rows_a, rows_b, sidx, didx, wc_a, wc_b, zb,
              sem_ga, sem_gb, sem_sa, sem_sb, sem_wa, sem_wb):
    c = lax.axis_index("c")
    s = lax.axis_index("s")

    pltpu.sync_copy(zrows, zb)

    def _zfill(g, _):
        r0 = pl.multiple_of(s * RPT + g * 32, 32)
        pltpu.sync_copy(zb, acc.at[pl.ds(r0, 32)])
        return _
    lax.fori_loop(0, RPT // 32, _zfill, 0)
    plsc.subcore_barrier()

    def _scale(buf, wc, base):
        def body(r8, _2):
            wr = jnp.int32(base) + r8
            for i in range(8):
                nb = wc[wr, pl.ds(i * LANES, LANES)]
                er = r8 * 8 + i
                for q in range(8):
                    seg = buf[er, pl.ds(q * LANES, LANES)]
                    buf[er, pl.ds(q * LANES, LANES)] = seg * nb
            return _2
        lax.fori_loop(0, 16, body, 0)

    gbufs = (rows_a, rows_b)
    gsems = (sem_ga, sem_gb)
    ssems = (sem_sa, sem_sb)
    wcs = (wc_a, wc_b)
    wsems = (sem_wa, sem_wb)

    def _chunk(g, _):
        row0 = pl.multiple_of(s * (EPT // 128) + g * 8, 8)
        pltpu.sync_copy(src2.at[pl.ds(row0, 8)], sidx)
        pltpu.sync_copy(dst2.at[pl.ds(row0, 8)], didx)
        wrow0 = pl.multiple_of(s * (EPT // 8) + g * 128, 8)
        wpend = [pltpu.async_copy(wb.at[pl.ds(wrow0, 32)], wc_a, sem_wa),
                 None]
        gpend = [pltpu.async_copy(h2.at[c].at[sidx.at[0]], rows_a, sem_ga),
                 None]
        spend = [None, None]
        for k in range(8):
            b = k % 2
            p = k // 2
            if k % 2 == 0:
                wpend[p % 2].wait()
                if p < 3:
                    wr1 = pl.multiple_of(wrow0 + (p + 1) * 32, 8)
                    wpend[(p + 1) % 2] = pltpu.async_copy(
                        wb.at[pl.ds(wr1, 32)], wcs[(p + 1) % 2],
                        wsems[(p + 1) % 2])
            gpend[b].wait()
            if k < 7:
                nb_ = (k + 1) % 2
                if spend[nb_] is not None:
                    spend[nb_].wait()
                gpend[nb_] = pltpu.async_copy(
                    h2.at[c].at[sidx.at[k + 1]], gbufs[nb_], gsems[nb_])
            _scale(gbufs[b], wcs[p % 2], (k % 2) * 16)
            spend[b] = pltpu.async_copy(gbufs[b], acc.at[didx.at[k]],
                                        ssems[b], add=True)
        spend[0].wait()
        spend[1].wait()
        return _
    lax.fori_loop(0, EPT // SC, _chunk, 0)
    plsc.subcore_barrier()

    nfull = N // RPT
    rem = N - nfull * RPT

    @pl.when(s < nfull)
    def _():
        r0 = pl.multiple_of(s * RPT, 64)
        pltpu.sync_copy(acc.at[pl.ds(r0, RPT)], out.at[c].at[pl.ds(r0, RPT)])

    @pl.when(s == nfull)
    def _():
        pltpu.sync_copy(acc.at[pl.ds(nfull * RPT, rem)],
                        out.at[c].at[pl.ds(nfull * RPT, rem)])


def _make_msg_kernel():
    return pl.kernel(
        _msg_body,
        out_type=jax.ShapeDtypeStruct((NCORE, N, DH), _f32),
        mesh=_mesh(),
        scratch_types=[
            pltpu.VMEM_SHARED((NPAD, DH), _f32),
            pltpu.VMEM((128, DH), _f32),
            pltpu.VMEM((128, DH), _f32),
            pltpu.VMEM((8, 128), _i32),
            pltpu.VMEM((8, 128), _i32),
            pltpu.VMEM((32, 128), _f32),
            pltpu.VMEM((32, 128), _f32),
            pltpu.VMEM((32, DH), _f32),
            pltpu.SemaphoreType.DMA,
            pltpu.SemaphoreType.DMA,
            pltpu.SemaphoreType.DMA,
            pltpu.SemaphoreType.DMA,
            pltpu.SemaphoreType.DMA,
            pltpu.SemaphoreType.DMA,
        ],
    )


def _rs_of(dref):
    return lax.rsqrt(jnp.maximum(dref[...], 1e-6))


def _t0_body(x_ref, w_ref, dsv_ref, o_ref):
    h = jnp.dot(x_ref[...], w_ref[...], preferred_element_type=_f32)
    h = h * _rs_of(dsv_ref)
    o_ref[0] = h[:, :DH]
    o_ref[1] = h[:, DH:]


def _tmid_body(emit_xin, a_ref, xin_ref, g_ref, be_ref, b_ref, w_ref,
               dsv_ref, ddv_ref, o_ref, *maybe_xin_out):
    rd = _rs_of(ddv_ref)
    rs = _rs_of(dsv_ref)
    acc = None
    cs = []
    for half in range(2):
        ch = a_ref[half] * rd + b_ref[half]
        cs.append(ch)
    for half in range(2):
        xin = cs[half] if emit_xin else xin_ref[half]
        t = jax.nn.relu(cs[half])
        mu = jnp.mean(t, axis=0, keepdims=True)
        var = jnp.mean((t - mu) ** 2, axis=0, keepdims=True)
        bn = g_ref[half] * (t - mu) / jnp.sqrt(var + 1e-5) + be_ref[half]
        u = xin + bn
        w = w_ref[pl.ds(half * DH, DH), :]
        part = jnp.dot(u, w, preferred_element_type=_f32)
        acc = part if acc is None else acc + part
    acc = acc * rs
    o_ref[0] = acc[:, :DH]
    o_ref[1] = acc[:, DH:]
    if emit_xin:
        xo_ref = maybe_xin_out[0]
        xo_ref[0] = cs[0]
        xo_ref[1] = cs[1]


def _tdeg_body(dsr_ref, ddr_ref, os_ref, od_ref):
    os_ref[...] = dsr_ref[0, :N, 0:1] + dsr_ref[1, :N, 0:1]
    od_ref[...] = ddr_ref[0, :N, 0:1] + ddr_ref[1, :N, 0:1]


def _tfin_body(a_ref, ddv_ref, b_ref, o_ref):
    rd = _rs_of(ddv_ref)
    o_ref[...] = jnp.concatenate(
        [a_ref[0] * rd + b_ref[0], a_ref[1] * rd + b_ref[1]], axis=1)


_TC_PARAMS = pltpu.CompilerParams(vmem_limit_bytes=100 * 1024 * 1024)

_t0 = pl.pallas_call(
    _t0_body, out_shape=jax.ShapeDtypeStruct((NCORE, N, DH), _f32),
    compiler_params=_TC_PARAMS)

_tmid0 = pl.pallas_call(
    functools.partial(_tmid_body, True),
    out_shape=(jax.ShapeDtypeStruct((NCORE, N, DH), _f32),
               jax.ShapeDtypeStruct((NCORE, N, DH), _f32)),
    compiler_params=_TC_PARAMS)

_tmid1 = pl.pallas_call(
    functools.partial(_tmid_body, False),
    out_shape=jax.ShapeDtypeStruct((NCORE, N, DH), _f32),
    compiler_params=_TC_PARAMS)

_tdeg = pl.pallas_call(
    _tdeg_body, out_shape=(jax.ShapeDtypeStruct((N, 1), _f32),
                           jax.ShapeDtypeStruct((N, 1), _f32)),
    compiler_params=_TC_PARAMS)

_tfin = pl.pallas_call(
    _tfin_body, out_shape=jax.ShapeDtypeStruct((N, D), _f32),
    compiler_params=_TC_PARAMS)


def kernel(x, edge_index, edge_attr, W0, b0, W1, b1, W2, b2,
           gamma0, beta0, gamma1, beta1):
    src = edge_index[0].astype(_i32)
    dst = edge_index[1].astype(_i32)
    pad = EP - E
    w_pad = jnp.pad(edge_attr, (0, pad))
    src2 = jnp.pad(src, (0, pad)).reshape(EP // 128, 128)
    dst2 = jnp.pad(dst, (0, pad)).reshape(EP // 128, 128)
    w16 = jnp.broadcast_to(w_pad[:, None], (EP, 16))
    wb = jnp.broadcast_to(w_pad[:, None], (EP, 16)).reshape(EP // 8, 128)
    zr16 = jnp.zeros((64, 16), _f32)
    zrows = jnp.zeros((32, DH), _f32)

    deg_k = _make_deg_kernel()
    msg_k = _make_msg_kernel()

    ds_raw, dd_raw = deg_k(src2, dst2, w16, zr16)
    dsv, ddv = _tdeg(ds_raw, dd_raw)
    b0_2 = b0.reshape(2, 1, DH)
    b1_2 = b1.reshape(2, 1, DH)
    b2_2 = b2.reshape(2, 1, DH)
    g0_2 = gamma0.reshape(2, 1, DH)
    be0_2 = beta0.reshape(2, 1, DH)
    g1_2 = gamma1.reshape(2, 1, DH)
    be1_2 = beta1.reshape(2, 1, DH)

    h0 = _t0(x, W0, dsv)
    a0 = msg_k(h0, src2, dst2, wb, zrows)
    h1, c0 = _tmid0(a0, a0, g0_2, be0_2, b0_2, W1, dsv, ddv)
    a1 = msg_k(h1, src2, dst2, wb, zrows)
    h2 = _tmid1(a1, c0, g1_2, be1_2, b1_2, W2, dsv, ddv)
    a2 = msg_k(h2, src2, dst2, wb, zrows)
    out = _tfin(a2, ddv, b2_2)
    return out
